# Initial kernel scaffold; baseline (speedup 1.0000x reference)
import jax
import jax.numpy as jnp
from jax.experimental import pallas as pl


def kernel(x, edge_index, W1, b1, W2, b2, Wenc, benc, Wfc, bfc):
    raise NotImplementedError("write your pallas kernel here")


# speedup vs baseline: 23.8105x; 23.8105x over previous
import functools

import jax
import jax.numpy as jnp
from jax import lax
from jax.experimental import pallas as pl
from jax.experimental.pallas import tpu as pltpu
from jax.experimental.pallas import tpu_sc as plsc

N = 10000
D = 128
E = 320000

NC = 2
NS = 16
NW = NC * NS
L = 16

N_PAD = 10240
TPB = N_PAD // NS
ZR = 64

CHUNK = 128
N_CH = 80
E_PAD = NW * N_CH * CHUNK

BLK = 1024
N_BLK = N_PAD // BLK


def _sc_mesh():
    return plsc.VectorSubcoreMesh(core_axis_name="c", subcore_axis_name="s")


_SC_PARAMS = pltpu.CompilerParams(needs_layout_passes=False)


def _deg_body(dst_hbm, out_hbm, hist, didx, tmp, acc, spart):
    c = lax.axis_index("c")
    s = lax.axis_index("s")
    wid = s * NC + c

    z16 = jnp.zeros((L,), jnp.float32)
    ones16 = jnp.ones((L,), jnp.float32)
    lanes = lax.iota(jnp.int32, L)

    def zero_hist(i, _):
        hist[pl.ds(i * L, L)] = z16
        return ()
    lax.fori_loop(0, N_PAD // L, zero_hist, ())

    def chunk(j, _):
        pltpu.sync_copy(dst_hbm.at[wid, j], didx)

        def sub(k, _):
            idx16 = didx[pl.ds(k * L, L)]
            for lane_i in range(L):
                plsc.addupdate_scatter(hist, [idx16], ones16,
                                       mask=lanes == lane_i)
            return ()
        return lax.fori_loop(0, CHUNK // L, sub, ())
    lax.fori_loop(0, N_CH, chunk, ())

    pltpu.sync_copy(hist, spart.at[s])
    plsc.subcore_barrier()

    base = s * TPB

    def zero_acc(i, _):
        acc[pl.ds(i * L, L)] = z16
        return ()
    lax.fori_loop(0, TPB // L, zero_acc, ())

    def red(t, _):
        pltpu.sync_copy(spart.at[t, pl.ds(base, TPB)], tmp)

        def add16(k, _):
            sl = pl.ds(k * L, L)
            acc[sl] = acc[sl] + tmp[sl]
            return ()
        return lax.fori_loop(0, TPB // L, add16, ())
    lax.fori_loop(0, NS, red, ())

    pltpu.sync_copy(acc, out_hbm.at[c, pl.ds(base, TPB)])


def _deg_kernel(dst_pad):
    return pl.kernel(
        _deg_body,
        out_type=jax.ShapeDtypeStruct((NC, N_PAD), jnp.float32),
        mesh=_sc_mesh(),
        compiler_params=_SC_PARAMS,
        scratch_types=[
            pltpu.VMEM((N_PAD,), jnp.float32),
            pltpu.VMEM((CHUNK,), jnp.int32),
            pltpu.VMEM((TPB,), jnp.float32),
            pltpu.VMEM((TPB,), jnp.float32),
            pltpu.VMEM_SHARED((NS, N_PAD), jnp.float32),
        ],
    )(dst_pad)


def _agg_body(hs_hbm, src_hbm, dst_hbm, out_hbm,
              acc, rows, sidx, didx, zbuf, sem0, sem1):
    c = lax.axis_index("c")
    s = lax.axis_index("s")
    wid = s * NC + c
    sems = (sem0, sem1)

    z16 = jnp.zeros((L,), jnp.float32)

    def zrow(i, _):
        def zcol(k, _):
            zbuf[i, pl.ds(k * L, L)] = z16
            return ()
        return lax.fori_loop(0, D // L, zcol, ())
    lax.fori_loop(0, ZR, zrow, ())

    def zacc(t, _):
        pltpu.sync_copy(zbuf, acc.at[pl.ds(s * TPB + t * ZR, ZR)])
        return ()
    lax.fori_loop(0, TPB // ZR, zacc, ())
    plsc.subcore_barrier()

    for b in range(2):
        pltpu.sync_copy(src_hbm.at[wid, b], sidx.at[b])
        pltpu.sync_copy(dst_hbm.at[wid, b], didx.at[b])
        pltpu.make_async_copy(hs_hbm.at[sidx.at[b]], rows.at[b],
                              sems[b]).start()

    def chunk_iter(g, _):
        for b in range(2):
            j = g * 2 + b
            pltpu.make_async_copy(hs_hbm.at[sidx.at[b]], rows.at[b],
                                  sems[b]).wait()
            pltpu.sync_copy(rows.at[b], acc.at[didx.at[b]], add=True)
            jn = j + 2

            @pl.when(jn < N_CH)
            def _prefetch():
                pltpu.sync_copy(src_hbm.at[wid, jn], sidx.at[b])
                pltpu.sync_copy(dst_hbm.at[wid, jn], didx.at[b])
                pltpu.make_async_copy(hs_hbm.at[sidx.at[b]], rows.at[b],
                                      sems[b]).start()
        return ()
    lax.fori_loop(0, N_CH // 2, chunk_iter, ())

    plsc.subcore_barrier()
    sl = pl.ds(s * TPB, TPB)
    pltpu.sync_copy(acc.at[sl], out_hbm.at[c, sl])


def _agg_kernel(hs, src_pad, dst_pad):
    return pl.kernel(
        _agg_body,
        out_type=jax.ShapeDtypeStruct((NC, N_PAD, D), jnp.float32),
        mesh=_sc_mesh(),
        compiler_params=_SC_PARAMS,
        scratch_types=[
            pltpu.VMEM_SHARED((N_PAD, D), jnp.float32),
            pltpu.VMEM((2, CHUNK, D), jnp.float32),
            pltpu.VMEM((2, CHUNK), jnp.int32),
            pltpu.VMEM((2, CHUNK), jnp.int32),
            pltpu.VMEM((ZR, D), jnp.float32),
            pltpu.SemaphoreType.DMA,
            pltpu.SemaphoreType.DMA,
        ],
    )(hs, src_pad, dst_pad)


def _enc_body(x_ref, w1_ref, wenc_ref, benc_ref, deg_ref, hs1_ref, xfc_ref):
    xb = x_ref[...]
    deg = deg_ref[0, :] + deg_ref[1, :] + 1.0
    dis = lax.rsqrt(deg)
    h1 = jnp.dot(xb, w1_ref[...], preferred_element_type=jnp.float32)
    hs1_ref[...] = h1 * dis[:, None]
    xfc = jnp.dot(xb, wenc_ref[...], preferred_element_type=jnp.float32)
    xfc_ref[...] = jnp.maximum(xfc + benc_ref[...], 0.0)


def _enc_kernel(x_pad, W1, Wenc, benc2, deg):
    return pl.pallas_call(
        _enc_body,
        grid=(N_BLK,),
        in_specs=[
            pl.BlockSpec((BLK, D), lambda i: (i, 0)),
            pl.BlockSpec((D, D), lambda i: (0, 0)),
            pl.BlockSpec((D, D), lambda i: (0, 0)),
            pl.BlockSpec((1, D), lambda i: (0, 0)),
            pl.BlockSpec((NC, BLK), lambda i: (0, i)),
        ],
        out_specs=[
            pl.BlockSpec((BLK, D), lambda i: (i, 0)),
            pl.BlockSpec((BLK, D), lambda i: (i, 0)),
        ],
        out_shape=[
            jax.ShapeDtypeStruct((N_PAD, D), jnp.float32),
            jax.ShapeDtypeStruct((N_PAD, D), jnp.float32),
        ],
    )(x_pad, W1, Wenc, benc2, deg)


def _mid_body(agg_ref, hs1_ref, xfc_ref, b1_ref, w2_ref, deg_ref,
              h_ref, hs2_ref):
    deg = deg_ref[0, :] + deg_ref[1, :] + 1.0
    dis = lax.rsqrt(deg)
    tot = agg_ref[0] + agg_ref[1] + hs1_ref[...]
    conv1 = jnp.maximum(tot * dis[:, None] + b1_ref[...], 0.0)
    h = conv1 + xfc_ref[...]
    h_ref[...] = h
    g = jnp.dot(h, w2_ref[...], preferred_element_type=jnp.float32)
    hs2_ref[...] = g * dis[:, None]


def _mid_kernel(agg1, hs1, xfc, b12, W2, deg):
    return pl.pallas_call(
        _mid_body,
        grid=(N_BLK,),
        in_specs=[
            pl.BlockSpec((NC, BLK, D), lambda i: (0, i, 0)),
            pl.BlockSpec((BLK, D), lambda i: (i, 0)),
            pl.BlockSpec((BLK, D), lambda i: (i, 0)),
            pl.BlockSpec((1, D), lambda i: (0, 0)),
            pl.BlockSpec((D, D), lambda i: (0, 0)),
            pl.BlockSpec((NC, BLK), lambda i: (0, i)),
        ],
        out_specs=[
            pl.BlockSpec((BLK, D), lambda i: (i, 0)),
            pl.BlockSpec((BLK, D), lambda i: (i, 0)),
        ],
        out_shape=[
            jax.ShapeDtypeStruct((N_PAD, D), jnp.float32),
            jax.ShapeDtypeStruct((N_PAD, D), jnp.float32),
        ],
    )(agg1, hs1, xfc, b12, W2, deg)


def _head_body(agg_ref, hs2_ref, h_ref, b2_ref, deg_ref, wfc_ref, bfc_ref,
               out_ref, acc_ref):
    i = pl.program_id(0)

    @pl.when(i == 0)
    def _init():
        acc_ref[...] = jnp.zeros_like(acc_ref)

    deg = deg_ref[0, :] + deg_ref[1, :] + 1.0
    dis = lax.rsqrt(deg)
    tot = agg_ref[0] + agg_ref[1] + hs2_ref[...]
    conv2 = jnp.maximum(tot * dis[:, None] + b2_ref[...], 0.0)
    h2 = conv2 + h_ref[...]
    rows = lax.broadcasted_iota(jnp.int32, (BLK, 1), 0) + i * BLK
    h2 = jnp.where(rows < N, h2, 0.0)
    acc_ref[...] = acc_ref[...] + jnp.sum(h2, axis=0, keepdims=True)

    @pl.when(i == N_BLK - 1)
    def _fin():
        pooled = acc_ref[...] / jnp.float32(N)
        logit = jnp.dot(pooled, wfc_ref[...],
                        preferred_element_type=jnp.float32) + bfc_ref[...]
        out_ref[...] = jax.nn.sigmoid(logit)


def _head_kernel(agg2, hs2, h, b22, deg, Wfc, bfc2):
    return pl.pallas_call(
        _head_body,
        grid=(N_BLK,),
        in_specs=[
            pl.BlockSpec((NC, BLK, D), lambda i: (0, i, 0)),
            pl.BlockSpec((BLK, D), lambda i: (i, 0)),
            pl.BlockSpec((BLK, D), lambda i: (i, 0)),
            pl.BlockSpec((1, D), lambda i: (0, 0)),
            pl.BlockSpec((NC, BLK), lambda i: (0, i)),
            pl.BlockSpec((D, 1), lambda i: (0, 0)),
            pl.BlockSpec((1, 1), lambda i: (0, 0)),
        ],
        out_specs=pl.BlockSpec((1, 1), lambda i: (0, 0)),
        out_shape=jax.ShapeDtypeStruct((1, 1), jnp.float32),
        scratch_shapes=[pltpu.VMEM((1, D), jnp.float32)],
    )(agg2, hs2, h, b22, deg, Wfc, bfc2)


def kernel(x, edge_index, W1, b1, W2, b2, Wenc, benc, Wfc, bfc):
    assert x.shape == (N, D) and edge_index.shape == (2, E)

    ei = edge_index.astype(jnp.int32)
    pad = N + (jnp.arange(E_PAD - E, dtype=jnp.int32) % (N_PAD - N))
    src_pad = jnp.concatenate([ei[0], pad]).reshape(NW, N_CH, CHUNK)
    dst_pad = jnp.concatenate([ei[1], pad]).reshape(NW, N_CH, CHUNK)
    x_pad = jnp.pad(x, ((0, N_PAD - N), (0, 0)))

    benc2 = benc.reshape(1, D)
    b12 = b1.reshape(1, D)
    b22 = b2.reshape(1, D)
    bfc2 = bfc.reshape(1, 1)

    deg = _deg_kernel(dst_pad)
    hs1, xfc = _enc_kernel(x_pad, W1, Wenc, benc2, deg)
    agg1 = _agg_kernel(hs1, src_pad, dst_pad)
    h, hs2 = _mid_kernel(agg1, hs1, xfc, b12, W2, deg)
    agg2 = _agg_kernel(hs2, src_pad, dst_pad)
    predict = _head_kernel(agg2, hs2, h, b22, deg, Wfc, bfc2)
    return predict


# speedup vs baseline: 29.0724x; 1.2210x over previous
import functools

import jax
import jax.numpy as jnp
from jax import lax
from jax.experimental import pallas as pl
from jax.experimental.pallas import tpu as pltpu
from jax.experimental.pallas import tpu_sc as plsc

N = 10000
D = 128
E = 320000

NC = 2
NS = 16
NW = NC * NS
L = 16

N_PAD = 10240
TPB = N_PAD // NS
ZR = 64

CHUNK = 128
N_CH = 80
E_PAD = NW * N_CH * CHUNK

BLK = 1024
N_BLK = N_PAD // BLK


def _sc_mesh():
    return plsc.VectorSubcoreMesh(core_axis_name="c", subcore_axis_name="s")


_SC_PARAMS = pltpu.CompilerParams(needs_layout_passes=False)


def _deg_body(dst_hbm, out_hbm, hist, didx, tmp, acc, spart):
    c = lax.axis_index("c")
    s = lax.axis_index("s")
    wid = s * NC + c

    z16 = jnp.zeros((L,), jnp.float32)

    def zero_hist(i, _):
        hist[pl.ds(i * L, L)] = z16
        return ()
    lax.fori_loop(0, N_PAD // L, zero_hist, ())

    pltpu.sync_copy(dst_hbm.at[wid], didx)

    def chunk(j, _):
        def sub(k, _):
            idx16 = didx[j, pl.ds(k * L, L)]
            cnt, last = plsc.scan_count(idx16)
            plsc.addupdate_scatter(hist, [idx16], cnt.astype(jnp.float32),
                                   mask=last)
            return ()
        return lax.fori_loop(0, CHUNK // L, sub, ())
    lax.fori_loop(0, N_CH, chunk, ())

    pltpu.sync_copy(hist, spart.at[s])
    plsc.subcore_barrier()

    base = s * TPB

    def zero_acc(i, _):
        acc[pl.ds(i * L, L)] = z16
        return ()
    lax.fori_loop(0, TPB // L, zero_acc, ())

    def red(t, _):
        pltpu.sync_copy(spart.at[t, pl.ds(base, TPB)], tmp)

        def add16(k, _):
            sl = pl.ds(k * L, L)
            acc[sl] = acc[sl] + tmp[sl]
            return ()
        return lax.fori_loop(0, TPB // L, add16, ())
    lax.fori_loop(0, NS, red, ())

    pltpu.sync_copy(acc, out_hbm.at[c, pl.ds(base, TPB)])


def _deg_kernel(dst_pad):
    return pl.kernel(
        _deg_body,
        out_type=jax.ShapeDtypeStruct((NC, N_PAD), jnp.float32),
        mesh=_sc_mesh(),
        compiler_params=_SC_PARAMS,
        scratch_types=[
            pltpu.VMEM((N_PAD,), jnp.float32),
            pltpu.VMEM((N_CH, CHUNK), jnp.int32),
            pltpu.VMEM((TPB,), jnp.float32),
            pltpu.VMEM((TPB,), jnp.float32),
            pltpu.VMEM_SHARED((NS, N_PAD), jnp.float32),
        ],
    )(dst_pad)


def _agg_body(hs_hbm, src_hbm, dst_hbm, out_hbm,
              acc, rows, sidx, dst_all, gsem0, gsem1, isem0, isem1):
    c = lax.axis_index("c")
    s = lax.axis_index("s")
    wid = s * NC + c
    gsems = (gsem0, gsem1)
    isems = (isem0, isem1)

    z16 = jnp.zeros((L,), jnp.float32)

    def zrow(i, _):
        def zcol(k, _):
            rows[0, i, pl.ds(k * L, L)] = z16
            return ()
        return lax.fori_loop(0, D // L, zcol, ())
    lax.fori_loop(0, CHUNK, zrow, ())

    def zacc(t, _):
        pltpu.sync_copy(rows.at[0],
                        acc.at[pl.ds(s * TPB + t * CHUNK, CHUNK)])
        return ()
    lax.fori_loop(0, TPB // CHUNK, zacc, ())

    pltpu.sync_copy(dst_hbm.at[wid], dst_all)
    plsc.subcore_barrier()

    pltpu.sync_copy(src_hbm.at[wid, 0], sidx.at[0])
    pltpu.make_async_copy(hs_hbm.at[sidx.at[0]], rows.at[0], gsems[0]).start()
    pltpu.make_async_copy(src_hbm.at[wid, 1], sidx.at[1], isems[1]).start()

    def chunk_iter(g, _):
        for b in range(2):
            j = g * 2 + b
            nb = 1 - b
            pltpu.make_async_copy(hs_hbm.at[sidx.at[b]], rows.at[b],
                                  gsems[b]).wait()

            @pl.when(j + 2 < N_CH)
            def _pre_idx():
                pltpu.make_async_copy(src_hbm.at[wid, j + 2], sidx.at[b],
                                      isems[b]).start()

            @pl.when(j + 1 < N_CH)
            def _pre_gather():
                pltpu.make_async_copy(src_hbm.at[wid, j + 1], sidx.at[nb],
                                      isems[nb]).wait()
                pltpu.make_async_copy(hs_hbm.at[sidx.at[nb]], rows.at[nb],
                                      gsems[nb]).start()

            pltpu.sync_copy(rows.at[b], acc.at[dst_all.at[j]], add=True)
        return ()
    lax.fori_loop(0, N_CH // 2, chunk_iter, ())

    plsc.subcore_barrier()
    sl = pl.ds(s * TPB, TPB)
    pltpu.sync_copy(acc.at[sl], out_hbm.at[c, sl])


def _agg_kernel(hs, src_pad, dst_pad):
    return pl.kernel(
        _agg_body,
        out_type=jax.ShapeDtypeStruct((NC, N_PAD, D), jnp.float32),
        mesh=_sc_mesh(),
        compiler_params=_SC_PARAMS,
        scratch_types=[
            pltpu.VMEM_SHARED((N_PAD, D), jnp.float32),
            pltpu.VMEM((2, CHUNK, D), jnp.float32),
            pltpu.VMEM((2, CHUNK), jnp.int32),
            pltpu.VMEM((N_CH, CHUNK), jnp.int32),
            pltpu.SemaphoreType.DMA,
            pltpu.SemaphoreType.DMA,
            pltpu.SemaphoreType.DMA,
            pltpu.SemaphoreType.DMA,
        ],
    )(hs, src_pad, dst_pad)


def _enc_body(x_ref, w1_ref, wenc_ref, benc_ref, deg_ref, hs1_ref, xfc_ref):
    xb = x_ref[...]
    deg = deg_ref[0, :] + deg_ref[1, :] + 1.0
    dis = lax.rsqrt(deg)
    h1 = jnp.dot(xb, w1_ref[...], preferred_element_type=jnp.float32)
    hs1_ref[...] = h1 * dis[:, None]
    xfc = jnp.dot(xb, wenc_ref[...], preferred_element_type=jnp.float32)
    xfc_ref[...] = jnp.maximum(xfc + benc_ref[...], 0.0)


def _enc_kernel(x_pad, W1, Wenc, benc2, deg):
    return pl.pallas_call(
        _enc_body,
        grid=(N_BLK,),
        in_specs=[
            pl.BlockSpec((BLK, D), lambda i: (i, 0)),
            pl.BlockSpec((D, D), lambda i: (0, 0)),
            pl.BlockSpec((D, D), lambda i: (0, 0)),
            pl.BlockSpec((1, D), lambda i: (0, 0)),
            pl.BlockSpec((NC, BLK), lambda i: (0, i)),
        ],
        out_specs=[
            pl.BlockSpec((BLK, D), lambda i: (i, 0)),
            pl.BlockSpec((BLK, D), lambda i: (i, 0)),
        ],
        out_shape=[
            jax.ShapeDtypeStruct((N_PAD, D), jnp.float32),
            jax.ShapeDtypeStruct((N_PAD, D), jnp.float32),
        ],
    )(x_pad, W1, Wenc, benc2, deg)


def _mid_body(agg_ref, hs1_ref, xfc_ref, b1_ref, w2_ref, deg_ref,
              h_ref, hs2_ref):
    deg = deg_ref[0, :] + deg_ref[1, :] + 1.0
    dis = lax.rsqrt(deg)
    tot = agg_ref[0] + agg_ref[1] + hs1_ref[...]
    conv1 = jnp.maximum(tot * dis[:, None] + b1_ref[...], 0.0)
    h = conv1 + xfc_ref[...]
    h_ref[...] = h
    g = jnp.dot(h, w2_ref[...], preferred_element_type=jnp.float32)
    hs2_ref[...] = g * dis[:, None]


def _mid_kernel(agg1, hs1, xfc, b12, W2, deg):
    return pl.pallas_call(
        _mid_body,
        grid=(N_BLK,),
        in_specs=[
            pl.BlockSpec((NC, BLK, D), lambda i: (0, i, 0)),
            pl.BlockSpec((BLK, D), lambda i: (i, 0)),
            pl.BlockSpec((BLK, D), lambda i: (i, 0)),
            pl.BlockSpec((1, D), lambda i: (0, 0)),
            pl.BlockSpec((D, D), lambda i: (0, 0)),
            pl.BlockSpec((NC, BLK), lambda i: (0, i)),
        ],
        out_specs=[
            pl.BlockSpec((BLK, D), lambda i: (i, 0)),
            pl.BlockSpec((BLK, D), lambda i: (i, 0)),
        ],
        out_shape=[
            jax.ShapeDtypeStruct((N_PAD, D), jnp.float32),
            jax.ShapeDtypeStruct((N_PAD, D), jnp.float32),
        ],
    )(agg1, hs1, xfc, b12, W2, deg)


def _head_body(agg_ref, hs2_ref, h_ref, b2_ref, deg_ref, wfc_ref, bfc_ref,
               out_ref, acc_ref):
    i = pl.program_id(0)

    @pl.when(i == 0)
    def _init():
        acc_ref[...] = jnp.zeros_like(acc_ref)

    deg = deg_ref[0, :] + deg_ref[1, :] + 1.0
    dis = lax.rsqrt(deg)
    tot = agg_ref[0] + agg_ref[1] + hs2_ref[...]
    conv2 = jnp.maximum(tot * dis[:, None] + b2_ref[...], 0.0)
    h2 = conv2 + h_ref[...]
    rows = lax.broadcasted_iota(jnp.int32, (BLK, 1), 0) + i * BLK
    h2 = jnp.where(rows < N, h2, 0.0)
    acc_ref[...] = acc_ref[...] + jnp.sum(h2, axis=0, keepdims=True)

    @pl.when(i == N_BLK - 1)
    def _fin():
        pooled = acc_ref[...] / jnp.float32(N)
        logit = jnp.dot(pooled, wfc_ref[...],
                        preferred_element_type=jnp.float32) + bfc_ref[...]
        out_ref[...] = jax.nn.sigmoid(logit)


def _head_kernel(agg2, hs2, h, b22, deg, Wfc, bfc2):
    return pl.pallas_call(
        _head_body,
        grid=(N_BLK,),
        in_specs=[
            pl.BlockSpec((NC, BLK, D), lambda i: (0, i, 0)),
            pl.BlockSpec((BLK, D), lambda i: (i, 0)),
            pl.BlockSpec((BLK, D), lambda i: (i, 0)),
            pl.BlockSpec((1, D), lambda i: (0, 0)),
            pl.BlockSpec((NC, BLK), lambda i: (0, i)),
            pl.BlockSpec((D, 1), lambda i: (0, 0)),
            pl.BlockSpec((1, 1), lambda i: (0, 0)),
        ],
        out_specs=pl.BlockSpec((1, 1), lambda i: (0, 0)),
        out_shape=jax.ShapeDtypeStruct((1, 1), jnp.float32),
        scratch_shapes=[pltpu.VMEM((1, D), jnp.float32)],
    )(agg2, hs2, h, b22, deg, Wfc, bfc2)


def kernel(x, edge_index, W1, b1, W2, b2, Wenc, benc, Wfc, bfc):
    assert x.shape == (N, D) and edge_index.shape == (2, E)

    ei = edge_index.astype(jnp.int32)
    pad = N + (jnp.arange(E_PAD - E, dtype=jnp.int32) % (N_PAD - N))
    src_pad = jnp.concatenate([ei[0], pad]).reshape(NW, N_CH, CHUNK)
    dst_pad = jnp.concatenate([ei[1], pad]).reshape(NW, N_CH, CHUNK)
    x_pad = jnp.pad(x, ((0, N_PAD - N), (0, 0)))

    benc2 = benc.reshape(1, D)
    b12 = b1.reshape(1, D)
    b22 = b2.reshape(1, D)
    bfc2 = bfc.reshape(1, 1)

    deg = _deg_kernel(dst_pad)
    hs1, xfc = _enc_kernel(x_pad, W1, Wenc, benc2, deg)
    agg1 = _agg_kernel(hs1, src_pad, dst_pad)
    h, hs2 = _mid_kernel(agg1, hs1, xfc, b12, W2, deg)
    agg2 = _agg_kernel(hs2, src_pad, dst_pad)
    predict = _head_kernel(agg2, hs2, h, b22, deg, Wfc, bfc2)
    return predict
